# baseline (speedup 1.0000x reference)
import dataclasses
import functools

import jax
import jax.numpy as jnp
from jax import lax
from jax.experimental import pallas as pl
from jax.experimental.pallas import tpu as pltpu
from jax.experimental.pallas import tpu_sc as plsc

_NC = 2
_NS = 16
_NW = _NC * _NS


def _embed_sc(idx, beT, goT, bpT, trT):
    B = idx.shape[0]
    d_be, n = beT.shape
    d_go, d_bp, d_tr = goT.shape[0], bpT.shape[0], trT.shape[0]
    rows_total = d_be + d_go + d_bp + d_tr
    slots = (rows_total + _NW - 1) // _NW
    e_go = d_be + d_go
    e_bp = e_go + d_bp
    mesh = plsc.VectorSubcoreMesh(core_axis_name="c", subcore_axis_name="s")
    cp = pltpu.CompilerParams()
    if "needs_layout_passes" in pltpu.CompilerParams.__dataclass_fields__:
        cp = dataclasses.replace(cp, needs_layout_passes=False)
    if "use_tc_tiling_on_sc" in pltpu.CompilerParams.__dataclass_fields__:
        cp = dataclasses.replace(cp, use_tc_tiling_on_sc=True)

    @functools.partial(
        pl.kernel,
        mesh=mesh,
        compiler_params=cp,
        out_type=(
            jax.ShapeDtypeStruct((d_be, B), beT.dtype),
            jax.ShapeDtypeStruct((d_go, B), goT.dtype),
            jax.ShapeDtypeStruct((d_bp, B), bpT.dtype),
            jax.ShapeDtypeStruct((d_tr, B), trT.dtype),
        ),
        scratch_types=[
            pltpu.VMEM((B,), jnp.int32),
            pltpu.VMEM((1, n), beT.dtype),
            pltpu.VMEM((1, 128), beT.dtype),
            pltpu.VMEM((B,), beT.dtype),
        ],
    )
    def k(beT_h, goT_h, bpT_h, trT_h, idx_h,
          obe_h, ogo_h, obp_h, otr_h,
          idx_v, row_v, bcol_v, orow_v):
        wid = lax.axis_index("s") * _NC + lax.axis_index("c")
        pltpu.sync_copy(idx_h, idx_v)
        zeros16 = lax.iota(jnp.int32, 16) * 0

        def gather_row(tbl_h, out_h, c):
            pltpu.sync_copy(tbl_h.at[pl.ds(c, 1)], row_v)

            @pl.loop(0, B, step=16)
            def _(o0):
                v = idx_v[pl.ds(o0, 16)]
                orow_v[pl.ds(o0, 16)] = plsc.load_gather(row_v, [zeros16, v])

            pltpu.sync_copy(orow_v, out_h.at[c])

        def bcast_row(c):
            pltpu.sync_copy(beT_h.at[pl.ds(c, 1), pl.ds(0, 128)], bcol_v)
            zv = lax.min(idx_v[pl.ds(0, 16)], 0)
            w = plsc.load_gather(bcol_v, [zv, zv])

            @pl.loop(0, B, step=16)
            def _(o0):
                orow_v[pl.ds(o0, 16)] = w

            pltpu.sync_copy(orow_v, obe_h.at[c])

        for s in range(slots):
            m = wid + _NW * s

            @pl.when(m < d_be)
            def _():
                bcast_row(m)

            @pl.when(jnp.logical_and(m >= d_be, m < e_go))
            def _():
                gather_row(goT_h, ogo_h, m - d_be)

            @pl.when(jnp.logical_and(m >= e_go, m < e_bp))
            def _():
                gather_row(bpT_h, obp_h, m - e_go)

            @pl.when(jnp.logical_and(m >= e_bp, m < rows_total))
            def _():
                gather_row(trT_h, otr_h, m - e_bp)

    return k(beT, goT, bpT, trT, idx)


def kernel(idx, betas, global_orient, body_pose, transl):
    idx = idx.astype(jnp.int32)
    obeT, ogoT, obpT, otrT = _embed_sc(
        idx, betas.T, global_orient.T, body_pose.T, transl.T)
    return (obeT.T, ogoT.T, obpT.T, otrT.T)
